# baseline (speedup 1.0000x reference)
import functools

import jax
import jax.numpy as jnp
from jax import lax
from jax.experimental import pallas as pl
from jax.experimental.pallas import tpu as pltpu
from jax.experimental.pallas import tpu_sc as plsc

TOPK = 8
NUM_CORES = 2
NUM_SUBCORES = 16
NUM_WORKERS = NUM_CORES * NUM_SUBCORES
LANES = 16


def _router_block(x_ref, w_ref, b_ref, o_ref):
    logits = lax.dot_general(
        w_ref[...], x_ref[...],
        (((1,), (1,)), ((), ())),
        preferred_element_type=jnp.float32,
    ) + b_ref[...]
    m = jnp.max(logits, axis=0, keepdims=True)
    e = jnp.exp(logits - m)
    o_ref[...] = e / jnp.sum(e, axis=0, keepdims=True)


@functools.partial(jax.jit, static_argnames=("row0", "nrows", "block_rows"))
def _router_tc(x, W, b, row0=0, nrows=None, block_rows=512):
    n, d = x.shape
    if nrows is None:
        nrows = n
    n_exp = W.shape[0]
    off = row0 // block_rows
    return pl.pallas_call(
        _router_block,
        grid=(nrows // block_rows,),
        in_specs=[
            pl.BlockSpec((block_rows, d), lambda i: (i + off, 0)),
            pl.BlockSpec((n_exp, d), lambda i: (0, 0)),
            pl.BlockSpec((n_exp, 1), lambda i: (0, 0)),
        ],
        out_specs=pl.BlockSpec((n_exp, block_rows), lambda i: (0, i)),
        out_shape=jax.ShapeDtypeStruct((n_exp, nrows), jnp.float32),
    )(x, W, b.reshape(n_exp, 1))


def _argmax_tree(vals):
    pairs = [(v, jnp.full((LANES,), e, jnp.int32)) for v, e in vals]
    while len(pairs) > 1:
        nxt = []
        for i in range(0, len(pairs), 2):
            (av, ai), (bv, bi) = pairs[i], pairs[i + 1]
            gt = bv > av
            nxt.append((jnp.where(gt, bv, av), jnp.where(gt, bi, ai)))
        pairs = nxt
    return pairs[0]


def _sc_gating(probs_t, n_exp):
    nc = probs_t.shape[1]
    rpw = nc // NUM_WORKERS
    groups = rpw // LANES
    mesh = plsc.VectorSubcoreMesh(
        core_axis_name="c", subcore_axis_name="s")

    @functools.partial(
        pl.kernel,
        out_type=jax.ShapeDtypeStruct((nc, n_exp), jnp.float32),
        mesh=mesh,
        compiler_params=pltpu.CompilerParams(needs_layout_passes=False),
        scratch_types=[
            pltpu.VMEM((n_exp, rpw), jnp.float32),
            pltpu.VMEM((rpw, n_exp), jnp.float32),
        ],
    )
    def gate(probs_hbm, out_hbm, pv, ov):
        wid = lax.axis_index("s") * NUM_CORES + lax.axis_index("c")
        pltpu.sync_copy(probs_hbm.at[:, pl.ds(wid * rpw, rpw)], pv)

        lane = lax.iota(jnp.int32, LANES)
        zero = jnp.zeros((LANES,), jnp.float32)
        neg = jnp.full((LANES,), -1.0, jnp.float32)

        def group_body(g, carry):
            col = g * LANES
            colv = col + lane
            for r in range(LANES):
                for j in range(n_exp // LANES):
                    ov[g * LANES + r, pl.ds(j * LANES, LANES)] = zero
            for _ in range(TOPK):
                vals = [(pv[e, pl.ds(col, LANES)], e)
                        for e in range(n_exp)]
                m, midx = _argmax_tree(vals)
                plsc.store_scatter(pv, [midx, colv], neg)
                plsc.store_scatter(ov, [colv, midx], m)
            return carry

        lax.fori_loop(0, groups, group_body, 0)
        pltpu.sync_copy(ov, out_hbm.at[pl.ds(wid * rpw, rpw), :])

    return gate(probs_t)


def _assemble(parts, n, n_exp):
    nc = parts[0].shape[0]

    def body(*refs):
        in_refs, (out_ref, sem) = refs[:-2], refs[-2:]
        copies = [
            pltpu.make_async_copy(
                in_refs[c], out_ref.at[pl.ds(c * nc, nc), :], sem)
            for c in range(len(in_refs))
        ]
        for cp in copies:
            cp.start()
        for cp in copies:
            cp.wait()

    return pl.pallas_call(
        body,
        in_specs=[pl.BlockSpec(memory_space=pl.ANY)] * len(parts),
        out_specs=pl.BlockSpec(memory_space=pl.ANY),
        out_shape=jax.ShapeDtypeStruct((n, n_exp), jnp.float32),
        scratch_shapes=[pltpu.SemaphoreType.DMA],
    )(*parts)


def kernel(x, W, b):
    n, _ = x.shape
    n_exp = W.shape[0]
    chunks = 4
    nc = n // chunks
    outs = []
    for c in range(chunks):
        probs_t = _router_tc(x, W, b, row0=c * nc, nrows=nc)
        outs.append(_sc_gating(probs_t, n_exp))
    return _assemble(outs, n, n_exp)


# speedup vs baseline: 2.8222x; 2.8222x over previous
import functools

import jax
import jax.numpy as jnp
from jax import lax
from jax.experimental import pallas as pl
from jax.experimental.pallas import tpu as pltpu
from jax.experimental.pallas import tpu_sc as plsc

TOPK = 8
NUM_CORES = 2
NUM_SUBCORES = 16
NUM_WORKERS = NUM_CORES * NUM_SUBCORES
LANES = 16


def _router_block(x_ref, w_ref, b_ref, o_ref):
    logits = lax.dot_general(
        w_ref[...], x_ref[...],
        (((1,), (1,)), ((), ())),
        preferred_element_type=jnp.float32,
    ) + b_ref[...]
    m = jnp.max(logits, axis=0, keepdims=True)
    e = jnp.exp(logits - m)
    o_ref[...] = e / jnp.sum(e, axis=0, keepdims=True)


@functools.partial(jax.jit, static_argnames=("row0", "nrows", "block_rows"))
def _router_tc(x, W, b, row0=0, nrows=None, block_rows=512):
    n, d = x.shape
    if nrows is None:
        nrows = n
    n_exp = W.shape[0]
    off = row0 // block_rows
    return pl.pallas_call(
        _router_block,
        grid=(nrows // block_rows,),
        in_specs=[
            pl.BlockSpec((block_rows, d), lambda i: (i + off, 0)),
            pl.BlockSpec((n_exp, d), lambda i: (0, 0)),
            pl.BlockSpec((n_exp, 1), lambda i: (0, 0)),
        ],
        out_specs=pl.BlockSpec((n_exp, block_rows), lambda i: (0, i)),
        out_shape=jax.ShapeDtypeStruct((n_exp, nrows), jnp.float32),
    )(x, W, b.reshape(n_exp, 1))


def _argmax_tree(vals):
    pairs = [(v, jnp.full((LANES,), e, jnp.int32)) for v, e in vals]
    while len(pairs) > 1:
        nxt = []
        for i in range(0, len(pairs), 2):
            (av, ai), (bv, bi) = pairs[i], pairs[i + 1]
            gt = bv > av
            nxt.append((jnp.where(gt, bv, av), jnp.where(gt, bi, ai)))
        pairs = nxt
    return pairs[0]


def _sc_gating(probs_t, prev, row0, n, n_exp):
    nc = probs_t.shape[1]
    rpw = nc // NUM_WORKERS
    groups = rpw // LANES
    cpr = row0 // NUM_WORKERS
    mesh = plsc.VectorSubcoreMesh(
        core_axis_name="c", subcore_axis_name="s")

    args = (probs_t,) if prev is None else (probs_t, prev)
    scratch = [
        pltpu.VMEM((n_exp, rpw), jnp.float32),
        pltpu.VMEM((rpw, n_exp), jnp.float32),
    ]
    if prev is not None:
        scratch += [pltpu.VMEM((cpr, n_exp), jnp.float32),
                    pltpu.SemaphoreType.DMA]

    @functools.partial(
        pl.kernel,
        out_type=jax.ShapeDtypeStruct((n, n_exp), jnp.float32),
        mesh=mesh,
        compiler_params=pltpu.CompilerParams(needs_layout_passes=False),
        scratch_types=scratch,
    )
    def gate(probs_hbm, *rest):
        if prev is None:
            (out_hbm, pv, ov) = rest
        else:
            (prev_hbm, out_hbm, pv, ov, cpv, csem) = rest
        wid = lax.axis_index("s") * NUM_CORES + lax.axis_index("c")
        if prev is not None:
            cp_in = pltpu.async_copy(
                prev_hbm.at[pl.ds(wid * cpr, cpr), :], cpv, csem)
        pltpu.sync_copy(probs_hbm.at[:, pl.ds(wid * rpw, rpw)], pv)

        lane = lax.iota(jnp.int32, LANES)
        zero = jnp.zeros((LANES,), jnp.float32)
        neg = jnp.full((LANES,), -1.0, jnp.float32)

        def group_body(g, carry):
            col = g * LANES
            colv = col + lane
            for r in range(LANES):
                for j in range(n_exp // LANES):
                    ov[g * LANES + r, pl.ds(j * LANES, LANES)] = zero
            for _ in range(TOPK):
                vals = [(pv[e, pl.ds(col, LANES)], e)
                        for e in range(n_exp)]
                m, midx = _argmax_tree(vals)
                plsc.store_scatter(pv, [midx, colv], neg)
                plsc.store_scatter(ov, [colv, midx], m)
            return carry

        lax.fori_loop(0, groups, group_body, 0)
        pltpu.sync_copy(ov, out_hbm.at[pl.ds(row0 + wid * rpw, rpw), :])
        if prev is not None:
            cp_in.wait()
            pltpu.sync_copy(cpv, out_hbm.at[pl.ds(wid * cpr, cpr), :])

    return gate(*args)


def kernel(x, W, b):
    n, _ = x.shape
    n_exp = W.shape[0]
    chunks = 4
    nc = n // chunks
    out = None
    for c in range(chunks):
        probs_t = _router_tc(x, W, b, row0=c * nc, nrows=nc)
        out = _sc_gating(probs_t, out, c * nc, n, n_exp)
    return out


# speedup vs baseline: 2.8240x; 1.0006x over previous
import functools

import jax
import jax.numpy as jnp
from jax import lax
from jax.experimental import pallas as pl
from jax.experimental.pallas import tpu as pltpu
from jax.experimental.pallas import tpu_sc as plsc

TOPK = 8
NUM_CORES = 2
NUM_SUBCORES = 16
NUM_WORKERS = NUM_CORES * NUM_SUBCORES
LANES = 16


def _router_block(x_ref, w_ref, b_ref, o_ref):
    logits = lax.dot_general(
        w_ref[...], x_ref[...],
        (((1,), (1,)), ((), ())),
        preferred_element_type=jnp.float32,
    ) + b_ref[...]
    m = jnp.max(logits, axis=0, keepdims=True)
    e = jnp.exp(logits - m)
    o_ref[...] = e / jnp.sum(e, axis=0, keepdims=True)


@functools.partial(jax.jit, static_argnames=("row0", "nrows", "block_rows"))
def _router_tc(x, W, b, row0=0, nrows=None, block_rows=512):
    n, d = x.shape
    if nrows is None:
        nrows = n
    n_exp = W.shape[0]
    off = row0 // block_rows
    return pl.pallas_call(
        _router_block,
        grid=(nrows // block_rows,),
        in_specs=[
            pl.BlockSpec((block_rows, d), lambda i: (i + off, 0)),
            pl.BlockSpec((n_exp, d), lambda i: (0, 0)),
            pl.BlockSpec((n_exp, 1), lambda i: (0, 0)),
        ],
        out_specs=pl.BlockSpec((n_exp, block_rows), lambda i: (0, i)),
        out_shape=jax.ShapeDtypeStruct((n_exp, nrows), jnp.float32),
    )(x, W, b.reshape(n_exp, 1))


def _argmax_tree(vals):
    pairs = [(v, jnp.full((LANES,), e, jnp.int32)) for v, e in vals]
    while len(pairs) > 1:
        nxt = []
        for i in range(0, len(pairs), 2):
            (av, ai), (bv, bi) = pairs[i], pairs[i + 1]
            gt = bv > av
            nxt.append((jnp.where(gt, bv, av), jnp.where(gt, bi, ai)))
        pairs = nxt
    return pairs[0]


def _sc_gating(probs_t, prev, row0, n, n_exp):
    nc = probs_t.shape[1]
    rpw = nc // NUM_WORKERS
    groups = rpw // LANES
    cpr = row0 // NUM_WORKERS
    mesh = plsc.VectorSubcoreMesh(
        core_axis_name="c", subcore_axis_name="s")

    args = (probs_t,) if prev is None else (probs_t, prev)
    scratch = [
        pltpu.VMEM((n_exp, rpw), jnp.float32),
        pltpu.VMEM((rpw, n_exp), jnp.float32),
    ]
    if prev is not None:
        scratch += [pltpu.VMEM((cpr, n_exp), jnp.float32),
                    pltpu.SemaphoreType.DMA]

    @functools.partial(
        pl.kernel,
        out_type=jax.ShapeDtypeStruct((n, n_exp), jnp.float32),
        mesh=mesh,
        compiler_params=pltpu.CompilerParams(needs_layout_passes=False),
        scratch_types=scratch,
    )
    def gate(probs_hbm, *rest):
        if prev is None:
            (out_hbm, pv, ov) = rest
        else:
            (prev_hbm, out_hbm, pv, ov, cpv, csem) = rest
        wid = lax.axis_index("s") * NUM_CORES + lax.axis_index("c")
        if prev is not None:
            cp_in = pltpu.async_copy(
                prev_hbm.at[pl.ds(wid * cpr, cpr), :], cpv, csem)
        pltpu.sync_copy(probs_hbm.at[:, pl.ds(wid * rpw, rpw)], pv)

        lane = lax.iota(jnp.int32, LANES)
        zero = jnp.zeros((LANES,), jnp.float32)
        neg = jnp.full((LANES,), -1.0, jnp.float32)

        def group_body(g, carry):
            col = g * LANES
            colv = col + lane
            for r in range(LANES):
                for j in range(n_exp // LANES):
                    ov[g * LANES + r, pl.ds(j * LANES, LANES)] = zero
            for _ in range(TOPK):
                vals = [(pv[e, pl.ds(col, LANES)], e)
                        for e in range(n_exp)]
                m, midx = _argmax_tree(vals)
                plsc.store_scatter(pv, [midx, colv], neg)
                plsc.store_scatter(ov, [colv, midx], m)
            return carry

        lax.fori_loop(0, groups, group_body, 0)
        pltpu.sync_copy(ov, out_hbm.at[pl.ds(row0 + wid * rpw, rpw), :])
        if prev is not None:
            cp_in.wait()
            pltpu.sync_copy(cpv, out_hbm.at[pl.ds(wid * cpr, cpr), :])

    return gate(*args)


def kernel(x, W, b):
    n, _ = x.shape
    n_exp = W.shape[0]
    chunks = 4
    nc = n // chunks
    out = None
    for c in range(chunks):
        probs_t = _router_tc(x, W, b, row0=c * nc, nrows=nc)
        out = _sc_gating(probs_t, out, c * nc, n, n_exp)
    return out + 0.0


# speedup vs baseline: 3.0538x; 1.0814x over previous
import functools

import jax
import jax.numpy as jnp
from jax import lax
from jax.experimental import pallas as pl
from jax.experimental.pallas import tpu as pltpu
from jax.experimental.pallas import tpu_sc as plsc

TOPK = 8
NUM_CORES = 2
NUM_SUBCORES = 16
NUM_WORKERS = NUM_CORES * NUM_SUBCORES
LANES = 16


def _router_block(x_ref, w_ref, b_ref, o_ref):
    logits = lax.dot_general(
        w_ref[...], x_ref[...],
        (((1,), (1,)), ((), ())),
        preferred_element_type=jnp.float32,
    ) + b_ref[...]
    m = jnp.max(logits, axis=0, keepdims=True)
    e = jnp.exp(logits - m)
    o_ref[...] = e / jnp.sum(e, axis=0, keepdims=True)


@functools.partial(jax.jit, static_argnames=("row0", "nrows", "block_rows"))
def _router_tc(x, W, b, row0=0, nrows=None, block_rows=512):
    n, d = x.shape
    if nrows is None:
        nrows = n
    n_exp = W.shape[0]
    off = row0 // block_rows
    return pl.pallas_call(
        _router_block,
        grid=(nrows // block_rows,),
        in_specs=[
            pl.BlockSpec((block_rows, d), lambda i: (i + off, 0)),
            pl.BlockSpec((n_exp, d), lambda i: (0, 0)),
            pl.BlockSpec((n_exp, 1), lambda i: (0, 0)),
        ],
        out_specs=pl.BlockSpec((n_exp, block_rows), lambda i: (0, i)),
        out_shape=jax.ShapeDtypeStruct((n_exp, nrows), jnp.float32),
    )(x, W, b.reshape(n_exp, 1))


def _argmax_tree(vals):
    pairs = [(v, jnp.full((LANES,), e, jnp.int32)) for v, e in vals]
    while len(pairs) > 1:
        nxt = []
        for i in range(0, len(pairs), 2):
            (av, ai), (bv, bi) = pairs[i], pairs[i + 1]
            gt = bv > av
            nxt.append((jnp.where(gt, bv, av), jnp.where(gt, bi, ai)))
        pairs = nxt
    return pairs[0]


def _sc_gating(probs_t, prev, row0, n, n_exp):
    nc = probs_t.shape[1]
    rpw = nc // NUM_WORKERS
    groups = rpw // LANES
    cpr = row0 // NUM_WORKERS
    mesh = plsc.VectorSubcoreMesh(
        core_axis_name="c", subcore_axis_name="s")

    args = (probs_t,) if prev is None else (probs_t, prev)
    scratch = [
        pltpu.VMEM((n_exp, rpw), jnp.float32),
        pltpu.VMEM((n_exp, rpw), jnp.float32),
    ]
    if prev is not None:
        scratch += [pltpu.VMEM((n_exp, cpr), jnp.float32),
                    pltpu.SemaphoreType.DMA]

    @functools.partial(
        pl.kernel,
        out_type=jax.ShapeDtypeStruct((n_exp, n), jnp.float32),
        mesh=mesh,
        compiler_params=pltpu.CompilerParams(needs_layout_passes=False),
        scratch_types=scratch,
    )
    def gate(probs_hbm, *rest):
        if prev is None:
            (out_hbm, pv, ov) = rest
        else:
            (prev_hbm, out_hbm, pv, ov, cpv, csem) = rest
        wid = lax.axis_index("s") * NUM_CORES + lax.axis_index("c")
        if prev is not None:
            cp_in = pltpu.async_copy(
                prev_hbm.at[:, pl.ds(wid * cpr, cpr)], cpv, csem)
        pltpu.sync_copy(probs_hbm.at[:, pl.ds(wid * rpw, rpw)], pv)

        lane = lax.iota(jnp.int32, LANES)
        zero = jnp.zeros((LANES,), jnp.float32)
        neg = jnp.full((LANES,), -1.0, jnp.float32)

        def group_body(g, carry):
            col = g * LANES
            colv = col + lane
            for e in range(n_exp):
                ov[e, pl.ds(col, LANES)] = zero
            for _ in range(TOPK):
                vals = [(pv[e, pl.ds(col, LANES)], e)
                        for e in range(n_exp)]
                m, midx = _argmax_tree(vals)
                plsc.store_scatter(pv, [midx, colv], neg)
                plsc.store_scatter(ov, [midx, colv], m)
            return carry

        lax.fori_loop(0, groups, group_body, 0)
        pltpu.sync_copy(ov, out_hbm.at[:, pl.ds(row0 + wid * rpw, rpw)])
        if prev is not None:
            cp_in.wait()
            pltpu.sync_copy(cpv, out_hbm.at[:, pl.ds(wid * cpr, cpr)])

    return gate(*args)


def kernel(x, W, b):
    n, _ = x.shape
    n_exp = W.shape[0]
    chunks = 4
    nc = n // chunks
    out = None
    for c in range(chunks):
        probs_t = _router_tc(x, W, b, row0=c * nc, nrows=nc)
        out = _sc_gating(probs_t, out, c * nc, n, n_exp)
    return out.T


# speedup vs baseline: 3.0821x; 1.0093x over previous
import functools

import jax
import jax.numpy as jnp
from jax import lax
from jax.experimental import pallas as pl
from jax.experimental.pallas import tpu as pltpu
from jax.experimental.pallas import tpu_sc as plsc

TOPK = 8
NUM_CORES = 2
NUM_SUBCORES = 16
NUM_WORKERS = NUM_CORES * NUM_SUBCORES
LANES = 16


def _router_block(x_ref, w_ref, b_ref, o_ref):
    logits = lax.dot_general(
        w_ref[...], x_ref[...],
        (((1,), (1,)), ((), ())),
        preferred_element_type=jnp.float32,
    ) + b_ref[...][:, None]
    m = jnp.max(logits, axis=0, keepdims=True)
    e = jnp.exp(logits - m)
    o_ref[...] = e / jnp.sum(e, axis=0, keepdims=True)


@functools.partial(jax.jit, static_argnames=("row0", "nrows", "block_rows"))
def _router_tc(x, W, b, row0=0, nrows=None, block_rows=512):
    n, d = x.shape
    if nrows is None:
        nrows = n
    n_exp = W.shape[0]
    off = row0 // block_rows
    return pl.pallas_call(
        _router_block,
        grid=(nrows // block_rows,),
        in_specs=[
            pl.BlockSpec((block_rows, d), lambda i: (i + off, 0)),
            pl.BlockSpec((n_exp, d), lambda i: (0, 0)),
            pl.BlockSpec((n_exp,), lambda i: (0,)),
        ],
        out_specs=pl.BlockSpec((n_exp, block_rows), lambda i: (0, i)),
        out_shape=jax.ShapeDtypeStruct((n_exp, nrows), jnp.float32),
    )(x, W, b)


def _argmax_tree(vals):
    pairs = [(v, jnp.full((LANES,), e, jnp.int32)) for v, e in vals]
    while len(pairs) > 1:
        nxt = []
        for i in range(0, len(pairs), 2):
            (av, ai), (bv, bi) = pairs[i], pairs[i + 1]
            gt = bv > av
            nxt.append((jnp.where(gt, bv, av), jnp.where(gt, bi, ai)))
        pairs = nxt
    return pairs[0]


def _sc_gating(probs_t, prev, row0, n, n_exp):
    nc = probs_t.shape[1]
    rpw = nc // NUM_WORKERS
    groups = rpw // LANES
    cpr = row0 // NUM_WORKERS
    mesh = plsc.VectorSubcoreMesh(
        core_axis_name="c", subcore_axis_name="s")

    args = (probs_t,) if prev is None else (probs_t, prev)
    scratch = [
        pltpu.VMEM((n_exp, rpw), jnp.float32),
        pltpu.VMEM((n_exp, rpw), jnp.float32),
    ]
    if prev is not None:
        scratch += [pltpu.VMEM((n_exp, cpr), jnp.float32),
                    pltpu.SemaphoreType.DMA]

    @functools.partial(
        pl.kernel,
        out_type=jax.ShapeDtypeStruct((n_exp, n), jnp.float32),
        mesh=mesh,
        compiler_params=pltpu.CompilerParams(needs_layout_passes=False),
        scratch_types=scratch,
    )
    def gate(probs_hbm, *rest):
        if prev is None:
            (out_hbm, pv, ov) = rest
        else:
            (prev_hbm, out_hbm, pv, ov, cpv, csem) = rest
        wid = lax.axis_index("s") * NUM_CORES + lax.axis_index("c")
        if prev is not None:
            cp_in = pltpu.async_copy(
                prev_hbm.at[:, pl.ds(wid * cpr, cpr)], cpv, csem)
        pltpu.sync_copy(probs_hbm.at[:, pl.ds(wid * rpw, rpw)], pv)

        lane = lax.iota(jnp.int32, LANES)
        zero = jnp.zeros((LANES,), jnp.float32)
        neg = jnp.full((LANES,), -1.0, jnp.float32)

        def group_body(g, carry):
            col = g * LANES
            colv = col + lane
            for e in range(n_exp):
                ov[e, pl.ds(col, LANES)] = zero
            for _ in range(TOPK):
                vals = [(pv[e, pl.ds(col, LANES)], e)
                        for e in range(n_exp)]
                m, midx = _argmax_tree(vals)
                plsc.store_scatter(pv, [midx, colv], neg)
                plsc.store_scatter(ov, [midx, colv], m)
            return carry

        lax.fori_loop(0, groups, group_body, 0)
        pltpu.sync_copy(ov, out_hbm.at[:, pl.ds(row0 + wid * rpw, rpw)])
        if prev is not None:
            cp_in.wait()
            pltpu.sync_copy(cpv, out_hbm.at[:, pl.ds(wid * cpr, cpr)])

    return gate(*args)


def kernel(x, W, b):
    n, _ = x.shape
    n_exp = W.shape[0]
    chunks = 4
    nc = n // chunks
    out = None
    for c in range(chunks):
        probs_t = _router_tc(x, W, b, row0=c * nc, nrows=nc)
        out = _sc_gating(probs_t, out, c * nc, n, n_exp)
    return out.T


# speedup vs baseline: 3.1012x; 1.0062x over previous
import functools

import jax
import jax.numpy as jnp
from jax import lax
from jax.experimental import pallas as pl
from jax.experimental.pallas import tpu as pltpu
from jax.experimental.pallas import tpu_sc as plsc

TOPK = 8
NUM_CORES = 2
NUM_SUBCORES = 16
NUM_WORKERS = NUM_CORES * NUM_SUBCORES
LANES = 16


def _router_block(x_ref, w_ref, b_ref, o_ref):
    logits = lax.dot_general(
        w_ref[...], x_ref[...],
        (((1,), (1,)), ((), ())),
        preferred_element_type=jnp.float32,
    ) + b_ref[...][:, None]
    m = jnp.max(logits, axis=0, keepdims=True)
    e = jnp.exp(logits - m)
    o_ref[...] = e / jnp.sum(e, axis=0, keepdims=True)


@functools.partial(jax.jit, static_argnames=("row0", "nrows", "block_rows"))
def _router_tc(x, W, b, row0=0, nrows=None, block_rows=512):
    n, d = x.shape
    if nrows is None:
        nrows = n
    n_exp = W.shape[0]
    off = row0 // block_rows
    return pl.pallas_call(
        _router_block,
        grid=(nrows // block_rows,),
        in_specs=[
            pl.BlockSpec((block_rows, d), lambda i: (i + off, 0)),
            pl.BlockSpec((n_exp, d), lambda i: (0, 0)),
            pl.BlockSpec((n_exp,), lambda i: (0,)),
        ],
        out_specs=pl.BlockSpec((n_exp, block_rows), lambda i: (0, i)),
        out_shape=jax.ShapeDtypeStruct((n_exp, nrows), jnp.float32),
    )(x, W, b)


def _argmax_tree(vals):
    pairs = [(v, jnp.full((LANES,), e, jnp.int32)) for v, e in vals]
    while len(pairs) > 1:
        nxt = []
        for i in range(0, len(pairs), 2):
            (av, ai), (bv, bi) = pairs[i], pairs[i + 1]
            gt = bv > av
            nxt.append((jnp.where(gt, bv, av), jnp.where(gt, bi, ai)))
        pairs = nxt
    return pairs[0]


def _sc_gating(probs_t, prev, row0, n, n_exp):
    nc = probs_t.shape[1]
    rpw = nc // NUM_WORKERS
    groups = rpw // LANES
    cpr = row0 // NUM_WORKERS
    mesh = plsc.VectorSubcoreMesh(
        core_axis_name="c", subcore_axis_name="s")

    args = (probs_t,) if prev is None else (probs_t, prev)
    scratch = [
        pltpu.VMEM((n_exp, rpw), jnp.float32),
        pltpu.VMEM((n_exp, rpw), jnp.float32),
        pltpu.SemaphoreType.DMA,
    ]
    if prev is not None:
        scratch += [pltpu.VMEM((n_exp, cpr), jnp.float32),
                    pltpu.SemaphoreType.DMA]

    @functools.partial(
        pl.kernel,
        out_type=jax.ShapeDtypeStruct((n_exp, n), jnp.float32),
        mesh=mesh,
        compiler_params=pltpu.CompilerParams(needs_layout_passes=False),
        scratch_types=scratch,
    )
    def gate(probs_hbm, *rest):
        if prev is None:
            (out_hbm, pv, ov, psem) = rest
        else:
            (prev_hbm, out_hbm, pv, ov, psem, cpv, csem) = rest
        wid = lax.axis_index("s") * NUM_CORES + lax.axis_index("c")
        if prev is not None:
            cp_in = pltpu.async_copy(
                prev_hbm.at[:, pl.ds(wid * cpr, cpr)], cpv, csem)
        p_in = pltpu.async_copy(
            probs_hbm.at[:, pl.ds(wid * rpw, rpw)], pv, psem)

        lane = lax.iota(jnp.int32, LANES)
        zero = jnp.zeros((LANES,), jnp.float32)
        neg = jnp.full((LANES,), -1.0, jnp.float32)

        def zero_body(g, carry):
            for e in range(n_exp):
                ov[e, pl.ds(g * LANES, LANES)] = zero
            return carry

        lax.fori_loop(0, groups, zero_body, 0)
        p_in.wait()

        def group_body(g, carry):
            col = g * LANES
            colv = col + lane
            for _ in range(TOPK):
                vals = [(pv[e, pl.ds(col, LANES)], e)
                        for e in range(n_exp)]
                m, midx = _argmax_tree(vals)
                plsc.store_scatter(pv, [midx, colv], neg)
                plsc.store_scatter(ov, [midx, colv], m)
            return carry

        lax.fori_loop(0, groups, group_body, 0)
        pltpu.sync_copy(ov, out_hbm.at[:, pl.ds(row0 + wid * rpw, rpw)])
        if prev is not None:
            cp_in.wait()
            pltpu.sync_copy(cpv, out_hbm.at[:, pl.ds(wid * cpr, cpr)])

    return gate(*args)


def kernel(x, W, b):
    n, _ = x.shape
    n_exp = W.shape[0]
    chunks = 4
    nc = n // chunks
    out = None
    for c in range(chunks):
        probs_t = _router_tc(x, W, b, row0=c * nc, nrows=nc)
        out = _sc_gating(probs_t, out, c * nc, n, n_exp)
    return out.T


# speedup vs baseline: 3.2149x; 1.0367x over previous
import functools

import jax
import jax.numpy as jnp
from jax import lax
from jax.experimental import pallas as pl
from jax.experimental.pallas import tpu as pltpu
from jax.experimental.pallas import tpu_sc as plsc

TOPK = 8
NUM_CORES = 2
NUM_SUBCORES = 16
NUM_WORKERS = NUM_CORES * NUM_SUBCORES
LANES = 16


def _router_block(x_ref, w_ref, b_ref, o_ref):
    logits = lax.dot_general(
        w_ref[...], x_ref[...],
        (((1,), (1,)), ((), ())),
        preferred_element_type=jnp.float32,
    ) + b_ref[...][:, None]
    m = jnp.max(logits, axis=0, keepdims=True)
    e = jnp.exp(logits - m)
    o_ref[...] = e / jnp.sum(e, axis=0, keepdims=True)


@functools.partial(jax.jit, static_argnames=("row0", "nrows", "block_rows"))
def _router_tc(x, W, b, row0=0, nrows=None, block_rows=512):
    n, d = x.shape
    if nrows is None:
        nrows = n
    n_exp = W.shape[0]
    off = row0 // block_rows
    return pl.pallas_call(
        _router_block,
        grid=(nrows // block_rows,),
        in_specs=[
            pl.BlockSpec((block_rows, d), lambda i: (i + off, 0)),
            pl.BlockSpec((n_exp, d), lambda i: (0, 0)),
            pl.BlockSpec((n_exp,), lambda i: (0,)),
        ],
        out_specs=pl.BlockSpec((n_exp, block_rows), lambda i: (0, i)),
        out_shape=jax.ShapeDtypeStruct((n_exp, nrows), jnp.float32),
    )(x, W, b)


def _argmax_tree(vals):
    pairs = [(v, jnp.full((LANES,), e, jnp.int32)) for v, e in vals]
    while len(pairs) > 1:
        nxt = []
        for i in range(0, len(pairs), 2):
            (av, ai), (bv, bi) = pairs[i], pairs[i + 1]
            gt = bv > av
            nxt.append((jnp.where(gt, bv, av), jnp.where(gt, bi, ai)))
        pairs = nxt
    return pairs[0]


def _sc_gating(probs_t, prev, row0, n, n_exp):
    nc = probs_t.shape[1]
    rpw = nc // NUM_WORKERS
    groups = rpw // LANES
    cpr = row0 // NUM_WORKERS
    mesh = plsc.VectorSubcoreMesh(
        core_axis_name="c", subcore_axis_name="s")

    args = (probs_t,) if prev is None else (probs_t, prev)
    scratch = [
        pltpu.VMEM((n_exp, rpw), jnp.float32),
        pltpu.VMEM((n_exp, rpw), jnp.float32),
        pltpu.SemaphoreType.DMA,
    ]
    if prev is not None:
        scratch += [pltpu.VMEM((n_exp, cpr), jnp.float32),
                    pltpu.SemaphoreType.DMA]

    @functools.partial(
        pl.kernel,
        out_type=jax.ShapeDtypeStruct((n_exp, n), jnp.float32),
        mesh=mesh,
        compiler_params=pltpu.CompilerParams(needs_layout_passes=False),
        scratch_types=scratch,
    )
    def gate(probs_hbm, *rest):
        if prev is None:
            (out_hbm, pv, ov, psem) = rest
        else:
            (prev_hbm, out_hbm, pv, ov, psem, cpv, csem) = rest
        wid = lax.axis_index("s") * NUM_CORES + lax.axis_index("c")
        if prev is not None:
            cp_in = pltpu.async_copy(
                prev_hbm.at[:, pl.ds(wid * cpr, cpr)], cpv, csem)
        p_in = pltpu.async_copy(
            probs_hbm.at[:, pl.ds(wid * rpw, rpw)], pv, psem)

        lane = lax.iota(jnp.int32, LANES)
        zero = jnp.zeros((LANES,), jnp.float32)
        neg = jnp.full((LANES,), -1.0, jnp.float32)

        def zero_body(g, carry):
            for e in range(n_exp):
                ov[e, pl.ds(g * LANES, LANES)] = zero
            return carry

        lax.fori_loop(0, groups, zero_body, 0)
        p_in.wait()

        def group_body(g, carry):
            col = g * LANES
            colv = col + lane
            for _ in range(TOPK):
                vals = [(pv[e, pl.ds(col, LANES)], e)
                        for e in range(n_exp)]
                m, midx = _argmax_tree(vals)
                plsc.store_scatter(pv, [midx, colv], neg)
                plsc.store_scatter(ov, [midx, colv], m)
            return carry

        lax.fori_loop(0, groups, group_body, 0)
        pltpu.sync_copy(ov, out_hbm.at[:, pl.ds(row0 + wid * rpw, rpw)])
        if prev is not None:
            cp_in.wait()
            pltpu.sync_copy(cpv, out_hbm.at[:, pl.ds(wid * cpr, cpr)])

    return gate(*args)


def kernel(x, W, b):
    n, _ = x.shape
    n_exp = W.shape[0]
    sizes = (n // 2, n // 4, n // 4)
    out = None
    row0 = 0
    for nc in sizes:
        probs_t = _router_tc(x, W, b, row0=row0, nrows=nc)
        out = _sc_gating(probs_t, out, row0, n, n_exp)
        row0 += nc
    return out.T


# speedup vs baseline: 3.3588x; 1.0448x over previous
import functools

import jax
import jax.numpy as jnp
from jax import lax
from jax.experimental import pallas as pl
from jax.experimental.pallas import tpu as pltpu
from jax.experimental.pallas import tpu_sc as plsc

TOPK = 8
NUM_CORES = 2
NUM_SUBCORES = 16
NUM_WORKERS = NUM_CORES * NUM_SUBCORES
LANES = 16


def _router_block(x_ref, w_ref, b_ref, o_ref):
    logits = lax.dot_general(
        w_ref[...], x_ref[...],
        (((1,), (1,)), ((), ())),
        preferred_element_type=jnp.float32,
    ) + b_ref[...][:, None]
    m = jnp.max(logits, axis=0, keepdims=True)
    e = jnp.exp(logits - m)
    o_ref[...] = e / jnp.sum(e, axis=0, keepdims=True)


@functools.partial(jax.jit, static_argnames=("row0", "nrows", "block_rows"))
def _router_tc(x, W, b, row0=0, nrows=None, block_rows=512):
    n, d = x.shape
    if nrows is None:
        nrows = n
    n_exp = W.shape[0]
    off = row0 // block_rows
    return pl.pallas_call(
        _router_block,
        grid=(nrows // block_rows,),
        in_specs=[
            pl.BlockSpec((block_rows, d), lambda i: (i + off, 0)),
            pl.BlockSpec((n_exp, d), lambda i: (0, 0)),
            pl.BlockSpec((n_exp,), lambda i: (0,)),
        ],
        out_specs=pl.BlockSpec((n_exp, block_rows), lambda i: (0, i)),
        out_shape=jax.ShapeDtypeStruct((n_exp, nrows), jnp.float32),
    )(x, W, b)


def _argmax_tree(vals):
    pairs = [(v, jnp.full((LANES,), e, jnp.int32)) for v, e in vals]
    while len(pairs) > 1:
        nxt = []
        for i in range(0, len(pairs), 2):
            (av, ai), (bv, bi) = pairs[i], pairs[i + 1]
            gt = bv > av
            nxt.append((jnp.where(gt, bv, av), jnp.where(gt, bi, ai)))
        pairs = nxt
    return pairs[0]


def _sc_gating(probs_t, prev, row0, n, n_exp):
    nc = probs_t.shape[1]
    rpw = nc // NUM_WORKERS
    groups = rpw // LANES
    cpr = row0 // NUM_WORKERS
    mesh = plsc.VectorSubcoreMesh(
        core_axis_name="c", subcore_axis_name="s")

    args = (probs_t,) if prev is None else (probs_t, prev)
    scratch = [
        pltpu.VMEM((n_exp, rpw), jnp.float32),
        pltpu.VMEM((n_exp, rpw), jnp.float32),
        pltpu.SemaphoreType.DMA,
    ]
    if prev is not None:
        scratch += [pltpu.VMEM((n_exp, cpr), jnp.float32),
                    pltpu.SemaphoreType.DMA]

    @functools.partial(
        pl.kernel,
        out_type=jax.ShapeDtypeStruct((n_exp, n), jnp.float32),
        mesh=mesh,
        compiler_params=pltpu.CompilerParams(needs_layout_passes=False),
        scratch_types=scratch,
    )
    def gate(probs_hbm, *rest):
        if prev is None:
            (out_hbm, pv, ov, psem) = rest
        else:
            (prev_hbm, out_hbm, pv, ov, psem, cpv, csem) = rest
        wid = lax.axis_index("s") * NUM_CORES + lax.axis_index("c")
        if prev is not None:
            cp_in = pltpu.async_copy(
                prev_hbm.at[:, pl.ds(wid * cpr, cpr)], cpv, csem)
        p_in = pltpu.async_copy(
            probs_hbm.at[:, pl.ds(wid * rpw, rpw)], pv, psem)

        lane = lax.iota(jnp.int32, LANES)
        zero = jnp.zeros((LANES,), jnp.float32)
        neg = jnp.full((LANES,), -1.0, jnp.float32)

        def zero_body(g, carry):
            for e in range(n_exp):
                ov[e, pl.ds(g * LANES, LANES)] = zero
            return carry

        lax.fori_loop(0, groups, zero_body, 0)
        p_in.wait()

        def group_body(g, carry):
            col = g * LANES
            colv = col + lane
            for _ in range(TOPK):
                vals = [(pv[e, pl.ds(col, LANES)], e)
                        for e in range(n_exp)]
                m, midx = _argmax_tree(vals)
                plsc.store_scatter(pv, [midx, colv], neg)
                plsc.store_scatter(ov, [midx, colv], m)
            return carry

        lax.fori_loop(0, groups, group_body, 0)
        pltpu.sync_copy(ov, out_hbm.at[:, pl.ds(row0 + wid * rpw, rpw)])
        if prev is not None:
            cp_in.wait()
            pltpu.sync_copy(cpv, out_hbm.at[:, pl.ds(wid * cpr, cpr)])

    return gate(*args)


def kernel(x, W, b):
    n, _ = x.shape
    n_exp = W.shape[0]
    sizes = (3 * n // 4, n // 4)
    out = None
    row0 = 0
    for nc in sizes:
        probs_t = _router_tc(x, W, b, row0=row0, nrows=nc)
        out = _sc_gating(probs_t, out, row0, n, n_exp)
        row0 += nc
    return out.T
